# baseline (speedup 1.0000x reference)
import functools

import jax
import jax.numpy as jnp
from jax import lax
from jax.experimental import pallas as pl
from jax.experimental.pallas import tpu as pltpu
from jax.experimental.pallas import tpu_sc as plsc

L = 16
ROWS = 8


def _sc_pe_add(B, S, D, MAXLEN):
    NC, NS = 2, 16
    NW = NC * NS
    sw = S // NW
    n_chunks = sw // ROWS

    mesh = plsc.VectorSubcoreMesh(core_axis_name="c", subcore_axis_name="s")

    @functools.partial(
        pl.kernel,
        out_type=jax.ShapeDtypeStruct((B, S, D), jnp.float32),
        mesh=mesh,
        scratch_types=[
            pltpu.VMEM((ROWS, D), jnp.float32),
            pltpu.VMEM((ROWS, D), jnp.float32),
            pltpu.VMEM((ROWS, D), jnp.float32),
        ],
    )
    def body(x_hbm, pe_hbm, out_hbm, pe_v, x_v, o_v):
        wid = lax.axis_index("s") * NC + lax.axis_index("c")
        base = wid * sw
        for kk in range(n_chunks):
            s0 = base + kk * ROWS
            pltpu.sync_copy(pe_hbm.at[pl.ds(s0, ROWS)], pe_v)
            for b in range(B):
                pltpu.sync_copy(x_hbm.at[b, pl.ds(s0, ROWS)], x_v)
                for r in range(ROWS):
                    def cbody(c, carry, r=r):
                        o_v[r, pl.ds(c * L, L)] = (
                            x_v[r, pl.ds(c * L, L)] + pe_v[r, pl.ds(c * L, L)]
                        )
                        return carry
                    lax.fori_loop(0, D // L, cbody, 0)
                pltpu.sync_copy(o_v, out_hbm.at[b, pl.ds(s0, ROWS)])

    return body


def kernel(x, pe_table):
    B, S, D = x.shape
    fn = _sc_pe_add(B, S, D, pe_table.shape[0])
    return fn(x, pe_table)
